# baseline (speedup 1.0000x reference)
import functools

import jax
import jax.numpy as jnp
from jax import lax
from jax.experimental import pallas as pl
from jax.experimental.pallas import tpu as pltpu
from jax.experimental.pallas import tpu_sc as plsc

N_SRC = 10000
N_DST = 10000
DEG = 16
D = 256
K = 8
L = 16
NC = 2
NS = 16
NW = NC * NS
E = N_DST * DEG
E_PER_W = E // NW
CHUNK = 8
N_CH = N_DST // CHUNK

_mesh = plsc.VectorSubcoreMesh(core_axis_name="c", subcore_axis_name="s")
_cparams = pltpu.CompilerParams(needs_layout_passes=False)


def _c(x):
    return jnp.int32(x)


def _wid():
    return lax.axis_index("s") * NC + lax.axis_index("c")


def _chunk_range(wid):
    base = N_CH // NW
    rem = N_CH % NW
    n_w = _c(base) + jnp.where(wid < _c(rem), _c(1), _c(0))
    s_w = wid * _c(base) + jnp.minimum(wid, _c(rem))
    return s_w, n_w


MAXCH_W = (N_CH + NW - 1) // NW
NBR_W = MAXCH_W * CHUNK * DEG
ROWS_CH = CHUNK * K
RIDX_PAD = ROWS_CH + L
SIMS_CH = CHUNK * DEG * DEG
ED_T = E // NS
SRC_W = 640
N_PAD = NS * SRC_W
HR = 4


@functools.partial(
    pl.kernel,
    mesh=_mesh,
    out_type=jax.ShapeDtypeStruct((N_DST, D), jnp.float32),
    compiler_params=_cparams,
    scratch_types=[
        pltpu.VMEM((ED_T,), jnp.int32),
        pltpu.VMEM((HR * N_SRC,), jnp.int32),
        pltpu.VMEM((NS * SRC_W,), jnp.int32),
        pltpu.VMEM((N_PAD,), jnp.float32),
        pltpu.VMEM((NBR_W,), jnp.int32),
        pltpu.VMEM((2 * SIMS_CH,), jnp.int32),
        pltpu.VMEM((2 * SIMS_CH,), jnp.float32),
        pltpu.VMEM((2 * RIDX_PAD,), jnp.int32),
        pltpu.VMEM((2 * RIDX_PAD,), jnp.float32),
        pltpu.VMEM((2 * ROWS_CH, D), jnp.float32),
        pltpu.VMEM((CHUNK, D), jnp.float32),
        pltpu.VMEM_SHARED((NS * N_PAD,), jnp.int32),
        pltpu.VMEM_SHARED((N_PAD,), jnp.float32),
        pltpu.SemaphoreType.DMA((2,)),
        pltpu.SemaphoreType.DMA((2,)),
    ],
)
def _fused_k(nbrf_hbm, sims_hbm, h_hbm, out_hbm,
             nbr_e, hist_v, part_l, norm_v, nbr_v, idx_v, s_v, ridx_v, w_v,
             rows_v, out_v, sh_part, sh_norm, sem_s, sem_r):
    wid = _wid()
    sid = lax.axis_index("s")
    iota = lax.iota(jnp.int32, L)

    pltpu.sync_copy(nbrf_hbm.at[pl.ds(sid * _c(ED_T), ED_T)], nbr_e)
    izero = jnp.zeros((L,), jnp.int32)
    rowoff = (iota & _c(HR - 1)) * _c(N_SRC)
    qmask = [(iota >> _c(2)) == _c(q) for q in range(4)]
    ones = jnp.full((L,), 1, jnp.int32)

    def zbody(i, carry):
        hist_v[pl.ds(i * _c(L), L)] = izero
        return carry

    lax.fori_loop(_c(0), _c(HR * N_SRC // L), zbody, _c(0))

    def ebody(t, carry):
        flat = rowoff + nbr_e[pl.ds(t * _c(L), L)]
        for q in range(4):
            plsc.addupdate_scatter(hist_v, [flat], ones, mask=qmask[q])
        return carry

    lax.fori_loop(_c(0), _c(ED_T // L), ebody, _c(0))

    def rbody(i, carry):
        acc = hist_v[pl.ds(i * _c(L), L)]
        for r in range(1, HR):
            acc = acc + hist_v[pl.ds(_c(r * N_SRC) + i * _c(L), L)]
        hist_v[pl.ds(i * _c(L), L)] = acc
        return carry

    lax.fori_loop(_c(0), _c(N_SRC // L), rbody, _c(0))
    row0 = pl.multiple_of(sid * _c(N_PAD), 128)
    pltpu.sync_copy(hist_v.at[pl.ds(0, N_PAD)], sh_part.at[pl.ds(row0, N_PAD)])
    plsc.subcore_barrier()

    s0 = pl.multiple_of(sid * _c(SRC_W), 128)
    cps = []
    for r in range(NS):
        cps.append(pltpu.async_copy(
            sh_part.at[pl.ds(_c(r * N_PAD) + s0, SRC_W)],
            part_l.at[pl.ds(r * SRC_W, SRC_W)], sem_s.at[_c(0)]))
    for cp in cps:
        cp.wait()

    def nbody(i, carry):
        o = i * _c(L)
        acc = part_l[pl.ds(o, L)]
        for r in range(1, NS):
            acc = acc + part_l[pl.ds(_c(r * SRC_W) + o, L)]
        x = jnp.maximum(acc.astype(jnp.float32), jnp.float32(1.0))
        yi = _c(0x5F3759DF) - (plsc.bitcast(x, jnp.int32) >> _c(1))
        y = plsc.bitcast(yi, jnp.float32)
        for _ in range(3):
            y = y * (jnp.float32(1.5) - jnp.float32(0.5) * x * y * y)
        norm_v[pl.ds(o, L)] = y
        return carry

    lax.fori_loop(_c(0), _c(SRC_W // L), nbody, _c(0))
    pltpu.sync_copy(norm_v.at[pl.ds(0, SRC_W)], sh_norm.at[pl.ds(s0, SRC_W)])
    plsc.subcore_barrier()
    pltpu.sync_copy(sh_norm, norm_v)

    s_w, n_w = _chunk_range(wid)
    s_pf = jnp.minimum(s_w, _c(N_CH - MAXCH_W))
    off = (s_w - s_pf) * _c(CHUNK * DEG)
    pltpu.sync_copy(nbrf_hbm.at[pl.ds(s_pf * _c(CHUNK * DEG), NBR_W)], nbr_v)

    iota = lax.iota(jnp.int32, L)
    lo8 = iota < _c(K)
    zero = jnp.zeros((L,), jnp.float32)
    izero = jnp.zeros((L,), jnp.int32)
    nlast = n_w - _c(1)

    def fetch_sims(g, par):
        loc = off + g * _c(CHUNK * DEG)
        pb = par * _c(SIMS_CH)
        for c in range(CHUNK):
            n_c = nbr_v[pl.ds(loc + _c(c * DEG), DEG)]
            n_scaled = n_c * _c(N_SRC)
            for i in range(DEG):
                idx_v[pl.ds(pb + _c(c * DEG * DEG + i * DEG), DEG)] = \
                    n_scaled[i] + n_c
        pltpu.async_copy(sims_hbm.at[idx_v.at[pl.ds(pb, SIMS_CH)]],
                         s_v.at[pl.ds(pb, SIMS_CH)], sem_s.at[par])

    def wait_sims(par):
        pb = par * _c(SIMS_CH)
        pltpu.make_async_copy(sims_hbm.at[idx_v.at[pl.ds(pb, SIMS_CH)]],
                              s_v.at[pl.ds(pb, SIMS_CH)],
                              sem_s.at[par]).wait()

    def greedy(g, par):
        loc = off + g * _c(CHUNK * DEG)
        pb = par * _c(SIMS_CH)
        wb = par * _c(RIDX_PAD)
        for c in range(CHUNK):
            base = pb + _c(c * DEG * DEG)
            rowsr = [s_v[pl.ds(base + _c(j * DEG), DEG)] for j in range(DEG)]
            cache = zero
            selvec = izero
            for t in range(K):
                gain = zero
                for j in range(DEG):
                    cjv = jnp.full((L,), cache[j], jnp.float32)
                    gain = gain + (jnp.maximum(rowsr[j], cjv) - cjv)
                m = jnp.max(gain)
                sel = plsc.all_reduce_ffs(gain == m)
                selrow = plsc.load_gather(
                    s_v, [base + sel * _c(DEG) + iota])
                cache = jnp.maximum(cache, selrow)
                selvec = jnp.where(iota == _c(t), sel, selvec)
            ids = plsc.load_gather(nbr_v, [loc + _c(c * DEG) + selvec])
            gw = plsc.load_gather(norm_v, [ids]) * jnp.float32(0.25)
            plsc.store_compressed(ridx_v.at[pl.ds(wb + _c(c * K), L)], ids,
                                  mask=lo8)
            plsc.store_compressed(w_v.at[pl.ds(wb + _c(c * K), L)], gw,
                                  mask=lo8)
        pltpu.async_copy(h_hbm.at[ridx_v.at[pl.ds(wb, ROWS_CH)]],
                         rows_v.at[pl.ds(par * _c(ROWS_CH), ROWS_CH)],
                         sem_r.at[par])

    def wait_rows(par):
        wb = par * _c(RIDX_PAD)
        pltpu.make_async_copy(
            h_hbm.at[ridx_v.at[pl.ds(wb, ROWS_CH)]],
            rows_v.at[pl.ds(par * _c(ROWS_CH), ROWS_CH)],
            sem_r.at[par]).wait()

    def agg_store(g, par):
        rb = par * _c(ROWS_CH)
        wb = par * _c(RIDX_PAD)
        for c in range(CHUNK):
            wvec = w_v[pl.ds(wb + _c(c * K), L)]
            wsp = []
            for t in range(K):
                wsp.append(jnp.full((L,), wvec[t], jnp.float32))
            for v in range(D // L):
                acc = wsp[0] * rows_v[rb + _c(c * K + 0), pl.ds(v * L, L)]
                for t in range(1, K):
                    acc = acc + wsp[t] * rows_v[rb + _c(c * K + t),
                                                pl.ds(v * L, L)]
                out_v[c, pl.ds(v * L, L)] = acc
        pltpu.sync_copy(out_v, out_hbm.at[pl.ds((s_w + g) * _c(CHUNK), CHUNK)])

    fetch_sims(_c(0), _c(0))

    def chunk_body(g, carry):
        par = jnp.bitwise_and(g, _c(1))
        parn = _c(1) - par
        fetch_sims(jnp.minimum(g + _c(1), nlast), parn)
        wait_sims(par)
        greedy(g, par)

        @pl.when(g > _c(0))
        def _():
            wait_rows(parn)
            agg_store(g - _c(1), parn)

        return carry

    lax.fori_loop(_c(0), n_w, chunk_body, _c(0))

    parl = jnp.bitwise_and(nlast, _c(1))
    wait_rows(parl)
    agg_store(nlast, parl)
    wait_sims(_c(1) - parl)


def kernel(h_src, h_dst, sims, neighbors, category):
    del h_dst, category
    nbr_flat = neighbors.astype(jnp.int32).reshape(-1)
    sims_flat = sims.reshape(-1)
    return _fused_k(nbr_flat, sims_flat, h_src)


# speedup vs baseline: 1.0196x; 1.0196x over previous
import functools

import jax
import jax.numpy as jnp
from jax import lax
from jax.experimental import pallas as pl
from jax.experimental.pallas import tpu as pltpu
from jax.experimental.pallas import tpu_sc as plsc

N_SRC = 10000
N_DST = 10000
DEG = 16
D = 256
K = 8
L = 16
NC = 2
NS = 16
NW = NC * NS
E = N_DST * DEG
E_PER_W = E // NW
CHUNK = 8
N_CH = N_DST // CHUNK

_mesh = plsc.VectorSubcoreMesh(core_axis_name="c", subcore_axis_name="s")
_cparams = pltpu.CompilerParams(needs_layout_passes=False)


def _c(x):
    return jnp.int32(x)


def _wid():
    return lax.axis_index("s") * NC + lax.axis_index("c")


def _chunk_range(wid):
    base = N_CH // NW
    rem = N_CH % NW
    n_w = _c(base) + jnp.where(wid < _c(rem), _c(1), _c(0))
    s_w = wid * _c(base) + jnp.minimum(wid, _c(rem))
    return s_w, n_w


@functools.partial(
    pl.kernel,
    mesh=_mesh,
    out_type=jax.ShapeDtypeStruct((NW, N_SRC), jnp.int32),
    compiler_params=_cparams,
    scratch_types=[
        pltpu.VMEM((E_PER_W,), jnp.int32),
        pltpu.VMEM((8 * N_SRC,), jnp.int32),
        pltpu.VMEM((N_SRC,), jnp.int32),
    ],
)
def _hist_k(nbr_hbm, out_hbm, nbr_v, hist8_v, hist_v):
    wid = _wid()
    pltpu.sync_copy(nbr_hbm.at[pl.ds(wid * _c(E_PER_W), E_PER_W)], nbr_v)

    zero = jnp.zeros((L,), jnp.int32)
    iota = lax.iota(jnp.int32, L)
    rowoff = (iota & _c(7)) * _c(N_SRC)
    lo = iota < _c(8)
    hi = jnp.logical_not(lo)
    ones = jnp.full((L,), 1, jnp.int32)

    def zbody(i, carry):
        hist8_v[pl.ds(i * _c(L), L)] = zero
        return carry

    lax.fori_loop(_c(0), _c(8 * N_SRC // L), zbody, _c(0))

    def ebody(t, carry):
        col = nbr_v[pl.ds(t * _c(L), L)]
        flat = rowoff + col
        plsc.addupdate_scatter(hist8_v, [flat], ones, mask=lo)
        plsc.addupdate_scatter(hist8_v, [flat], ones, mask=hi)
        return carry

    lax.fori_loop(_c(0), _c(E_PER_W // L), ebody, _c(0))

    def rbody(i, carry):
        acc = hist8_v[pl.ds(i * _c(L), L)]
        for r in range(1, 8):
            acc = acc + hist8_v[pl.ds(_c(r * N_SRC) + i * _c(L), L)]
        hist_v[pl.ds(i * _c(L), L)] = acc
        return carry

    lax.fori_loop(_c(0), _c(N_SRC // L), rbody, _c(0))
    pltpu.sync_copy(hist_v, out_hbm.at[wid])


def _norm_body(hist_ref, out_ref):
    deg = jnp.sum(hist_ref[...].astype(jnp.float32), axis=0, keepdims=True,
                  dtype=jnp.float32)
    out_ref[...] = lax.rsqrt(jnp.maximum(deg, jnp.float32(1.0)))


_norm_call = pl.pallas_call(
    _norm_body,
    out_shape=jax.ShapeDtypeStruct((1, N_SRC), jnp.float32),
)


MAXCH_W = (N_CH + NW - 1) // NW
NBR_W = MAXCH_W * CHUNK * DEG
ROWS_CH = CHUNK * K
RIDX_PAD = ROWS_CH + L
SIMS_CH = CHUNK * DEG * DEG


@functools.partial(
    pl.kernel,
    mesh=_mesh,
    out_type=jax.ShapeDtypeStruct((N_DST, D), jnp.float32),
    compiler_params=_cparams,
    scratch_types=[
        pltpu.VMEM((N_SRC,), jnp.float32),
        pltpu.VMEM((NBR_W,), jnp.int32),
        pltpu.VMEM((2 * SIMS_CH,), jnp.int32),
        pltpu.VMEM((2 * SIMS_CH,), jnp.float32),
        pltpu.VMEM((2 * RIDX_PAD,), jnp.int32),
        pltpu.VMEM((2 * RIDX_PAD,), jnp.float32),
        pltpu.VMEM((2 * ROWS_CH, D), jnp.float32),
        pltpu.VMEM((CHUNK, D), jnp.float32),
        pltpu.SemaphoreType.DMA((2,)),
        pltpu.SemaphoreType.DMA((2,)),
    ],
)
def _selagg_k(sims_hbm, nbrf_hbm, norm_hbm, h_hbm, out_hbm,
              norm_v, nbr_v, idx_v, s_v, ridx_v, w_v, rows_v, out_v,
              sem_s, sem_r):
    wid = _wid()
    pltpu.sync_copy(norm_hbm, norm_v)
    s_w, n_w = _chunk_range(wid)
    s_pf = jnp.minimum(s_w, _c(N_CH - MAXCH_W))
    off = (s_w - s_pf) * _c(CHUNK * DEG)
    pltpu.sync_copy(nbrf_hbm.at[pl.ds(s_pf * _c(CHUNK * DEG), NBR_W)], nbr_v)

    iota = lax.iota(jnp.int32, L)
    lo8 = iota < _c(K)
    zero = jnp.zeros((L,), jnp.float32)
    izero = jnp.zeros((L,), jnp.int32)
    nlast = n_w - _c(1)

    def fetch_sims(g, par):
        loc = off + g * _c(CHUNK * DEG)
        pb = par * _c(SIMS_CH)
        for c in range(CHUNK):
            n_c = nbr_v[pl.ds(loc + _c(c * DEG), DEG)]
            n_scaled = n_c * _c(N_SRC)
            for i in range(DEG):
                idx_v[pl.ds(pb + _c(c * DEG * DEG + i * DEG), DEG)] = \
                    n_scaled[i] + n_c
        pltpu.async_copy(sims_hbm.at[idx_v.at[pl.ds(pb, SIMS_CH)]],
                         s_v.at[pl.ds(pb, SIMS_CH)], sem_s.at[par])

    def wait_sims(par):
        pb = par * _c(SIMS_CH)
        pltpu.make_async_copy(sims_hbm.at[idx_v.at[pl.ds(pb, SIMS_CH)]],
                              s_v.at[pl.ds(pb, SIMS_CH)],
                              sem_s.at[par]).wait()

    def greedy(g, par):
        loc = off + g * _c(CHUNK * DEG)
        pb = par * _c(SIMS_CH)
        wb = par * _c(RIDX_PAD)
        for c in range(CHUNK):
            base = pb + _c(c * DEG * DEG)
            rowsr = [s_v[pl.ds(base + _c(j * DEG), DEG)] for j in range(DEG)]
            cache = zero
            selvec = izero
            for t in range(K):
                gain = zero
                for j in range(DEG):
                    cjv = jnp.full((L,), cache[j], jnp.float32)
                    gain = gain + (jnp.maximum(rowsr[j], cjv) - cjv)
                m = jnp.max(gain)
                sel = plsc.all_reduce_ffs(gain == m)
                selrow = plsc.load_gather(
                    s_v, [base + sel * _c(DEG) + iota])
                cache = jnp.maximum(cache, selrow)
                selvec = jnp.where(iota == _c(t), sel, selvec)
            ids = plsc.load_gather(nbr_v, [loc + _c(c * DEG) + selvec])
            gw = plsc.load_gather(norm_v, [ids]) * jnp.float32(0.25)
            plsc.store_compressed(ridx_v.at[pl.ds(wb + _c(c * K), L)], ids,
                                  mask=lo8)
            plsc.store_compressed(w_v.at[pl.ds(wb + _c(c * K), L)], gw,
                                  mask=lo8)
        pltpu.async_copy(h_hbm.at[ridx_v.at[pl.ds(wb, ROWS_CH)]],
                         rows_v.at[pl.ds(par * _c(ROWS_CH), ROWS_CH)],
                         sem_r.at[par])

    def wait_rows(par):
        wb = par * _c(RIDX_PAD)
        pltpu.make_async_copy(
            h_hbm.at[ridx_v.at[pl.ds(wb, ROWS_CH)]],
            rows_v.at[pl.ds(par * _c(ROWS_CH), ROWS_CH)],
            sem_r.at[par]).wait()

    def agg_store(g, par):
        rb = par * _c(ROWS_CH)
        wb = par * _c(RIDX_PAD)
        for c in range(CHUNK):
            wvec = w_v[pl.ds(wb + _c(c * K), L)]
            wsp = []
            for t in range(K):
                wsp.append(jnp.full((L,), wvec[t], jnp.float32))
            for v in range(D // L):
                acc = wsp[0] * rows_v[rb + _c(c * K + 0), pl.ds(v * L, L)]
                for t in range(1, K):
                    acc = acc + wsp[t] * rows_v[rb + _c(c * K + t),
                                                pl.ds(v * L, L)]
                out_v[c, pl.ds(v * L, L)] = acc
        pltpu.sync_copy(out_v, out_hbm.at[pl.ds((s_w + g) * _c(CHUNK), CHUNK)])

    fetch_sims(_c(0), _c(0))

    def chunk_body(g, carry):
        par = jnp.bitwise_and(g, _c(1))
        parn = _c(1) - par
        fetch_sims(jnp.minimum(g + _c(1), nlast), parn)
        wait_sims(par)
        greedy(g, par)

        @pl.when(g > _c(0))
        def _():
            wait_rows(parn)
            agg_store(g - _c(1), parn)

        return carry

    lax.fori_loop(_c(0), n_w, chunk_body, _c(0))

    parl = jnp.bitwise_and(nlast, _c(1))
    wait_rows(parl)
    agg_store(nlast, parl)
    wait_sims(_c(1) - parl)


def kernel(h_src, h_dst, sims, neighbors, category):
    del h_dst, category
    nbr_flat = neighbors.astype(jnp.int32).reshape(-1)
    sims_flat = sims.reshape(-1)
    hist = _hist_k(nbr_flat)
    norm = _norm_call(hist).reshape(N_SRC)
    out = _selagg_k(sims_flat, nbr_flat, norm, h_src)
    return out
